# Initial kernel scaffold; baseline (speedup 1.0000x reference)
import jax
import jax.numpy as jnp
from jax.experimental import pallas as pl


def kernel(x, edge_index, fc_in_w, fc_in_b, lin_l_w, lin_l_b, lin_r_w, norm_weight, norm_bias, norm_mean_scale, fc_out_w, fc_out_b):
    raise NotImplementedError("write your pallas kernel here")


# speedup vs baseline: 6.8167x; 6.8167x over previous
import functools

import jax
import jax.numpy as jnp
from jax import lax
from jax.experimental import pallas as pl
from jax.experimental.pallas import tpu as pltpu
from jax.experimental.pallas import tpu_sc as plsc

N = 10000
D = 128
NPAD = 10240
E = 320000
NC, NS = 2, 16
NW = NC * NS
CHUNK = 128
EPW = ((E // NW + CHUNK - 1) // CHUNK) * CHUNK
EPAD = EPW * NW
NCHUNK = EPW // CHUNK
RPT = NPAD // NS


def _zero_f32(ref, n):
    def body(i, _):
        ref[pl.ds(i * 16, 16)] = jnp.zeros((16,), jnp.float32)
        return 0
    lax.fori_loop(0, n // 16, body, 0)


def _sc_aggregate(h_hbm, src_hbm, dst_hbm, acc_out, cnt_out,
                  acc_sh, cnt_sh, rows_v, sidx_v, didx_v, ones_v, cbuf_v, sem):
    cid = lax.axis_index("c")
    sid = lax.axis_index("s")
    wid = cid * NS + sid

    def zrow(r, _):
        for c in range(D // 16):
            rows_v[r, pl.ds(c * 16, 16)] = jnp.zeros((16,), jnp.float32)
        return 0
    lax.fori_loop(0, CHUNK, zrow, 0)
    _zero_f32(cbuf_v, RPT)

    def zones(i, _):
        ones_v[pl.ds(i * 16, 16)] = jnp.ones((16,), jnp.float32)
        return 0
    lax.fori_loop(0, CHUNK // 16, zones, 0)

    for k in range(RPT // CHUNK):
        pltpu.sync_copy(rows_v, acc_sh.at[pl.ds(sid * RPT + k * CHUNK, CHUNK)])
    pltpu.sync_copy(cbuf_v, cnt_sh.at[pl.ds(sid * RPT, RPT)])
    plsc.subcore_barrier()

    def edge_body(j, _):
        base = pl.multiple_of(wid * EPW + j * CHUNK, CHUNK)
        pltpu.sync_copy(src_hbm.at[pl.ds(base, CHUNK)], sidx_v)
        pltpu.sync_copy(dst_hbm.at[pl.ds(base, CHUNK)], didx_v)
        pltpu.async_copy(h_hbm.at[sidx_v], rows_v, sem).wait()
        pltpu.sync_copy(rows_v, acc_sh.at[didx_v], add=True)
        pltpu.sync_copy(ones_v, cnt_sh.at[didx_v], add=True)
        return 0
    lax.fori_loop(0, NCHUNK, edge_body, 0)

    plsc.subcore_barrier()

    for k in range(RPT // CHUNK):
        r0 = sid * RPT + k * CHUNK
        pltpu.sync_copy(acc_sh.at[pl.ds(r0, CHUNK)], rows_v)
        pltpu.sync_copy(rows_v, acc_out.at[pl.ds(cid * NPAD + r0, CHUNK)])
    pltpu.sync_copy(cnt_sh.at[pl.ds(sid * RPT, RPT)], cbuf_v)
    pltpu.sync_copy(cbuf_v, cnt_out.at[pl.ds(cid * NPAD + sid * RPT, RPT)])


_sc_agg_call = functools.partial(
    pl.kernel,
    out_type=(
        jax.ShapeDtypeStruct((NC * NPAD, D), jnp.float32),
        jax.ShapeDtypeStruct((NC * NPAD,), jnp.float32),
    ),
    mesh=plsc.VectorSubcoreMesh(
        core_axis_name="c", subcore_axis_name="s", num_cores=NC, num_subcores=NS
    ),
    scratch_types=[
        pltpu.VMEM_SHARED((NPAD, D), jnp.float32),
        pltpu.VMEM_SHARED((NPAD,), jnp.float32),
        pltpu.VMEM((CHUNK, D), jnp.float32),
        pltpu.VMEM((CHUNK,), jnp.int32),
        pltpu.VMEM((CHUNK,), jnp.int32),
        pltpu.VMEM((CHUNK,), jnp.float32),
        pltpu.VMEM((RPT,), jnp.float32),
        pltpu.SemaphoreType.DMA,
    ],
)(_sc_aggregate)


def _fc_in_body(x_ref, w_ref, b_ref, o_ref):
    o_ref[...] = lax.dot_general(
        x_ref[...], w_ref[...], (((1,), (1,)), ((), ())),
        preferred_element_type=jnp.float32) + b_ref[...]


def _tail_body(acc_ref, cnt_ref, h_ref, wl_ref, bl_ref, wr_ref,
               nw_ref, nb_ref, alpha_ref, wo_ref, bo_ref, o_ref):
    p = acc_ref[0:N, :] + acc_ref[NPAD:NPAD + N, :]
    c = cnt_ref[0:N, :] + cnt_ref[NPAD:NPAD + N, :]
    mean = p / jnp.clip(c, 1.0, None)
    h = h_ref[0:N, :]
    h2 = (lax.dot_general(mean, wl_ref[...], (((1,), (1,)), ((), ())),
                          preferred_element_type=jnp.float32)
          + bl_ref[...]
          + lax.dot_general(h, wr_ref[...], (((1,), (1,)), ((), ())),
                            preferred_element_type=jnp.float32))
    mu = jnp.mean(h2, axis=0, keepdims=True)
    centered = h2 - alpha_ref[...] * mu
    var = jnp.mean(centered * centered, axis=0, keepdims=True)
    hn = nw_ref[...] * (centered * lax.rsqrt(var + 1e-5)) + nb_ref[...]
    ha = jnp.where(hn > 0, hn, 0.1 * hn)
    o_ref[...] = lax.dot_general(
        ha, wo_ref[...], (((1,), (1,)), ((), ())),
        preferred_element_type=jnp.float32) + bo_ref[...]


def kernel(x, edge_index, fc_in_w, fc_in_b, lin_l_w, lin_l_b, lin_r_w,
           norm_weight, norm_bias, norm_mean_scale, fc_out_w, fc_out_b):
    f32 = jnp.float32
    xp = jnp.pad(x, ((0, NPAD - N), (0, 0)))
    npad_e = EPAD - E
    sent = (N + (jnp.arange(npad_e, dtype=jnp.int32) % (NPAD - N))).astype(jnp.int32)
    src = jnp.concatenate([edge_index[0], sent])
    dst = jnp.concatenate([edge_index[1], sent])

    h = pl.pallas_call(
        _fc_in_body,
        out_shape=jax.ShapeDtypeStruct((NPAD, D), f32),
    )(xp, fc_in_w, fc_in_b[None, :])

    acc, cnt = _sc_agg_call(h, src, dst)

    out = pl.pallas_call(
        _tail_body,
        out_shape=jax.ShapeDtypeStruct((N, D), f32),
    )(acc, cnt[:, None], h, lin_l_w, lin_l_b[None, :], lin_r_w,
      norm_weight[None, :], norm_bias[None, :], norm_mean_scale[None, :],
      fc_out_w, fc_out_b[None, :])
    return out


# speedup vs baseline: 9.5613x; 1.4026x over previous
import functools

import jax
import jax.numpy as jnp
from jax import lax
from jax.experimental import pallas as pl
from jax.experimental.pallas import tpu as pltpu
from jax.experimental.pallas import tpu_sc as plsc

N = 10000
D = 128
NPAD = 10240
E = 320000
NC, NS = 2, 16
NW = NC * NS
CHUNK = 64
K = 4
HALVES = 4
GRAN = CHUNK * K * HALVES
EPW = ((E // NW + GRAN - 1) // GRAN) * GRAN
EPAD = EPW * NW
NCHUNK = EPW // CHUNK
NCH = NCHUNK // HALVES
RPT = NPAD // NS


def _zero_f32(ref, n):
    def body(i, _):
        ref[pl.ds(i * 16, 16)] = jnp.zeros((16,), jnp.float32)
        return 0
    lax.fori_loop(0, n // 16, body, 0)


def _sc_aggregate(h_hbm, src_hbm, dst_hbm, acc_out, cnt_out,
                  acc_sh, cnt_sh, rows_v, sidx_v, didx_v, ones_v, cbuf_v,
                  gsem, ssem, csem):
    cid = lax.axis_index("c")
    sid = lax.axis_index("s")
    wid = cid * NS + sid
    base_w = pl.multiple_of(wid * EPW, CHUNK)

    def zrow(r, _):
        for c in range(D // 16):
            rows_v[0, r, pl.ds(c * 16, 16)] = jnp.zeros((16,), jnp.float32)
        return 0
    lax.fori_loop(0, CHUNK, zrow, 0)
    _zero_f32(cbuf_v, RPT)

    def zones(i, _):
        ones_v[pl.ds(i * 16, 16)] = jnp.ones((16,), jnp.float32)
        return 0
    lax.fori_loop(0, CHUNK // 16, zones, 0)

    for k in range(RPT // CHUNK):
        pltpu.sync_copy(rows_v.at[0],
                        acc_sh.at[pl.ds(sid * RPT + k * CHUNK, CHUNK)])
    pltpu.sync_copy(cbuf_v, cnt_sh.at[pl.ds(sid * RPT, RPT)])
    plsc.subcore_barrier()

    def edge_body(g, _):
        j0 = g * K
        gathers = []
        for b in range(K):
            idx = sidx_v.at[pl.ds((j0 + b) * CHUNK, CHUNK)]
            gathers.append(pltpu.async_copy(h_hbm.at[idx], rows_v.at[b], gsem))
        for cp in gathers:
            cp.wait()
        scat = []
        for b in range(K):
            didx = didx_v.at[j0 + b]
            scat.append(pltpu.async_copy(rows_v.at[b], acc_sh.at[didx], ssem,
                                         add=True))
            scat.append(pltpu.async_copy(ones_v, cnt_sh.at[didx], csem,
                                         add=True))
        for cp in scat:
            cp.wait()
        return 0

    for half in range(HALVES):
        pltpu.sync_copy(
            src_hbm.at[pl.ds(base_w + half * (EPW // HALVES), EPW // HALVES)],
            sidx_v)
        pltpu.sync_copy(
            dst_hbm.at[pl.ds(wid * NCHUNK + half * NCH, NCH)], didx_v)
        lax.fori_loop(0, NCH // K, edge_body, 0)

    plsc.subcore_barrier()

    for k in range(RPT // CHUNK):
        r0 = sid * RPT + k * CHUNK
        pltpu.sync_copy(acc_sh.at[pl.ds(r0, CHUNK)], rows_v.at[0])
        pltpu.sync_copy(rows_v.at[0], acc_out.at[pl.ds(cid * NPAD + r0, CHUNK)])
    pltpu.sync_copy(cnt_sh.at[pl.ds(sid * RPT, RPT)], cbuf_v)
    pltpu.sync_copy(cbuf_v, cnt_out.at[pl.ds(cid * NPAD + sid * RPT, RPT)])


_sc_agg_call = functools.partial(
    pl.kernel,
    out_type=(
        jax.ShapeDtypeStruct((NC * NPAD, D), jnp.float32),
        jax.ShapeDtypeStruct((NC * NPAD,), jnp.float32),
    ),
    mesh=plsc.VectorSubcoreMesh(
        core_axis_name="c", subcore_axis_name="s", num_cores=NC, num_subcores=NS
    ),
    scratch_types=[
        pltpu.VMEM_SHARED((NPAD, D), jnp.float32),
        pltpu.VMEM_SHARED((NPAD,), jnp.float32),
        pltpu.VMEM((K, CHUNK, D), jnp.float32),
        pltpu.VMEM((EPW // HALVES,), jnp.int32),
        pltpu.VMEM((NCH, CHUNK), jnp.int32),
        pltpu.VMEM((CHUNK,), jnp.float32),
        pltpu.VMEM((RPT,), jnp.float32),
        pltpu.SemaphoreType.DMA,
        pltpu.SemaphoreType.DMA,
        pltpu.SemaphoreType.DMA,
    ],
)(_sc_aggregate)


def _fc_in_body(x_ref, w_ref, b_ref, o_ref):
    o_ref[...] = lax.dot_general(
        x_ref[...], w_ref[...], (((1,), (1,)), ((), ())),
        preferred_element_type=jnp.float32) + b_ref[...]


def _tail_body(acc_ref, cnt_ref, h_ref, wl_ref, bl_ref, wr_ref,
               nw_ref, nb_ref, alpha_ref, wo_ref, bo_ref, o_ref):
    p = acc_ref[0:N, :] + acc_ref[NPAD:NPAD + N, :]
    c = cnt_ref[0:N, :] + cnt_ref[NPAD:NPAD + N, :]
    mean = p / jnp.clip(c, 1.0, None)
    h = h_ref[0:N, :]
    h2 = (lax.dot_general(mean, wl_ref[...], (((1,), (1,)), ((), ())),
                          preferred_element_type=jnp.float32)
          + bl_ref[...]
          + lax.dot_general(h, wr_ref[...], (((1,), (1,)), ((), ())),
                            preferred_element_type=jnp.float32))
    mu = jnp.mean(h2, axis=0, keepdims=True)
    centered = h2 - alpha_ref[...] * mu
    var = jnp.mean(centered * centered, axis=0, keepdims=True)
    hn = nw_ref[...] * (centered * lax.rsqrt(var + 1e-5)) + nb_ref[...]
    ha = jnp.where(hn > 0, hn, 0.1 * hn)
    o_ref[...] = lax.dot_general(
        ha, wo_ref[...], (((1,), (1,)), ((), ())),
        preferred_element_type=jnp.float32) + bo_ref[...]


def kernel(x, edge_index, fc_in_w, fc_in_b, lin_l_w, lin_l_b, lin_r_w,
           norm_weight, norm_bias, norm_mean_scale, fc_out_w, fc_out_b):
    f32 = jnp.float32
    xp = jnp.pad(x, ((0, NPAD - N), (0, 0)))
    npad_e = EPAD - E
    sent = (N + (jnp.arange(npad_e, dtype=jnp.int32) % (NPAD - N))).astype(jnp.int32)
    src = jnp.concatenate([edge_index[0], sent])
    dst = jnp.concatenate([edge_index[1], sent]).reshape(EPAD // CHUNK, CHUNK)

    h = pl.pallas_call(
        _fc_in_body,
        out_shape=jax.ShapeDtypeStruct((NPAD, D), f32),
    )(xp, fc_in_w, fc_in_b[None, :])

    acc, cnt = _sc_agg_call(h, src, dst)

    out = pl.pallas_call(
        _tail_body,
        out_shape=jax.ShapeDtypeStruct((N, D), f32),
    )(acc, cnt[:, None], h, lin_l_w, lin_l_b[None, :], lin_r_w,
      norm_weight[None, :], norm_bias[None, :], norm_mean_scale[None, :],
      fc_out_w, fc_out_b[None, :])
    return out


# speedup vs baseline: 9.7891x; 1.0238x over previous
import functools

import jax
import jax.numpy as jnp
from jax import lax
from jax.experimental import pallas as pl
from jax.experimental.pallas import tpu as pltpu
from jax.experimental.pallas import tpu_sc as plsc

N = 10000
D = 128
NPAD = 10240
E = 320000
NC, NS = 2, 16
NW = NC * NS
CHUNK = 64
K = 4
HALVES = 4
GRAN = CHUNK * K * HALVES
EPW = ((E // NW + GRAN - 1) // GRAN) * GRAN
EPAD = EPW * NW
NCHUNK = EPW // CHUNK
NCH = NCHUNK // HALVES
RPT = NPAD // NS


def _zero_f32(ref, n):
    def body(i, _):
        ref[pl.ds(i * 16, 16)] = jnp.zeros((16,), jnp.float32)
        return 0
    lax.fori_loop(0, n // 16, body, 0)


def _sc_aggregate(x_hbm, src_hbm, dst_hbm, acc_out, cnt_out,
                  acc_sh, cnt_sh, rows_v, sidx_v, didx_v, ones_v, cbuf_v,
                  gsem, ssem, csem):
    cid = lax.axis_index("c")
    sid = lax.axis_index("s")
    wid = cid * NS + sid
    base_w = pl.multiple_of(wid * EPW, CHUNK)

    def zrow(r, _):
        for c in range(D // 16):
            rows_v[0, r, pl.ds(c * 16, 16)] = jnp.zeros((16,), jnp.float32)
        return 0
    lax.fori_loop(0, CHUNK, zrow, 0)
    _zero_f32(cbuf_v, RPT)

    def zones(i, _):
        ones_v[pl.ds(i * 16, 16)] = jnp.ones((16,), jnp.float32)
        return 0
    lax.fori_loop(0, CHUNK // 16, zones, 0)

    for k in range(RPT // CHUNK):
        pltpu.sync_copy(rows_v.at[0],
                        acc_sh.at[pl.ds(sid * RPT + k * CHUNK, CHUNK)])
    pltpu.sync_copy(cbuf_v, cnt_sh.at[pl.ds(sid * RPT, RPT)])
    plsc.subcore_barrier()

    def edge_body(g, _):
        j0 = g * K
        gathers = []
        for b in range(K):
            idx = sidx_v.at[pl.ds((j0 + b) * CHUNK, CHUNK)]
            gathers.append(pltpu.async_copy(x_hbm.at[idx], rows_v.at[b], gsem))
        for cp in gathers:
            cp.wait()
        scat = []
        for b in range(K):
            didx = didx_v.at[j0 + b]
            scat.append(pltpu.async_copy(rows_v.at[b], acc_sh.at[didx], ssem,
                                         add=True))
            scat.append(pltpu.async_copy(ones_v, cnt_sh.at[didx], csem,
                                         add=True))
        for cp in scat:
            cp.wait()
        return 0

    for half in range(HALVES):
        pltpu.sync_copy(
            src_hbm.at[pl.ds(base_w + half * (EPW // HALVES), EPW // HALVES)],
            sidx_v)
        pltpu.sync_copy(
            dst_hbm.at[pl.ds(wid * NCHUNK + half * NCH, NCH)], didx_v)
        lax.fori_loop(0, NCH // K, edge_body, 0)

    plsc.subcore_barrier()

    pltpu.sync_copy(acc_sh.at[pl.ds(sid * RPT, RPT)],
                    acc_out.at[pl.ds(cid * NPAD + sid * RPT, RPT)])
    pltpu.sync_copy(cnt_sh.at[pl.ds(sid * RPT, RPT)],
                    cnt_out.at[pl.ds(cid * NPAD + sid * RPT, RPT)])


_sc_agg_call = functools.partial(
    pl.kernel,
    out_type=(
        jax.ShapeDtypeStruct((NC * NPAD, D), jnp.float32),
        jax.ShapeDtypeStruct((NC * NPAD,), jnp.float32),
    ),
    mesh=plsc.VectorSubcoreMesh(
        core_axis_name="c", subcore_axis_name="s", num_cores=NC, num_subcores=NS
    ),
    scratch_types=[
        pltpu.VMEM_SHARED((NPAD, D), jnp.float32),
        pltpu.VMEM_SHARED((NPAD,), jnp.float32),
        pltpu.VMEM((K, CHUNK, D), jnp.float32),
        pltpu.VMEM((EPW // HALVES,), jnp.int32),
        pltpu.VMEM((NCH, CHUNK), jnp.int32),
        pltpu.VMEM((CHUNK,), jnp.float32),
        pltpu.VMEM((RPT,), jnp.float32),
        pltpu.SemaphoreType.DMA,
        pltpu.SemaphoreType.DMA,
        pltpu.SemaphoreType.DMA,
    ],
)(_sc_aggregate)


def _tail_body(acc_ref, cnt_ref, x_ref, wi_ref, bi_ref, wl_ref, bl_ref,
               wr_ref, nw_ref, nb_ref, alpha_ref, wo_ref, bo_ref, o_ref):
    p = acc_ref[0:N, :] + acc_ref[NPAD:NPAD + N, :]
    c = cnt_ref[0:N, :] + cnt_ref[NPAD:NPAD + N, :]
    cc = jnp.clip(c, 1.0, None)
    meanx = p / cc
    mask = c / cc
    mm = lambda a, b: lax.dot_general(a, b, (((1,), (1,)), ((), ())),
                                      preferred_element_type=jnp.float32)
    w1 = mm(wl_ref[...], wi_ref[...].T)
    w2 = mm(wr_ref[...], wi_ref[...].T)
    bi = bi_ref[...]
    h2 = (mm(meanx, w1) + mm(x_ref[0:N, :], w2)
          + mask * mm(bi, wl_ref[...]) + mm(bi, wr_ref[...])
          + bl_ref[...])
    mu = jnp.mean(h2, axis=0, keepdims=True)
    centered = h2 - alpha_ref[...] * mu
    var = jnp.mean(centered * centered, axis=0, keepdims=True)
    hn = nw_ref[...] * (centered * lax.rsqrt(var + 1e-5)) + nb_ref[...]
    ha = jnp.where(hn > 0, hn, 0.1 * hn)
    o_ref[...] = mm(ha, wo_ref[...]) + bo_ref[...]


def kernel(x, edge_index, fc_in_w, fc_in_b, lin_l_w, lin_l_b, lin_r_w,
           norm_weight, norm_bias, norm_mean_scale, fc_out_w, fc_out_b):
    f32 = jnp.float32
    xp = jnp.pad(x, ((0, NPAD - N), (0, 0)))
    npad_e = EPAD - E
    sent = (N + (jnp.arange(npad_e, dtype=jnp.int32) % (NPAD - N))).astype(jnp.int32)
    src = jnp.concatenate([edge_index[0], sent])
    dst = jnp.concatenate([edge_index[1], sent]).reshape(EPAD // CHUNK, CHUNK)

    acc, cnt = _sc_agg_call(xp, src, dst)

    out = pl.pallas_call(
        _tail_body,
        out_shape=jax.ShapeDtypeStruct((N, D), f32),
    )(acc, cnt[:, None], xp, fc_in_w, fc_in_b[None, :], lin_l_w,
      lin_l_b[None, :], lin_r_w, norm_weight[None, :], norm_bias[None, :],
      norm_mean_scale[None, :], fc_out_w, fc_out_b[None, :])
    return out


# speedup vs baseline: 11.9806x; 1.2239x over previous
import functools

import jax
import jax.numpy as jnp
from jax import lax
from jax.experimental import pallas as pl
from jax.experimental.pallas import tpu as pltpu
from jax.experimental.pallas import tpu_sc as plsc

N = 10000
D = 128
NPAD = 10240
E = 320000
NC, NS = 2, 16
NW = NC * NS
CHUNK = 64
K = 2
NB = 2
HALVES = 4
GRAN = CHUNK * K * HALVES
EPW = ((E // NW + GRAN - 1) // GRAN) * GRAN
EPAD = EPW * NW
NCHUNK = EPW // CHUNK
NCH = NCHUNK // HALVES
RPT = NPAD // NS


def _zero_f32(ref, n):
    def body(i, _):
        ref[pl.ds(i * 16, 16)] = jnp.zeros((16,), jnp.float32)
        return 0
    lax.fori_loop(0, n // 16, body, 0)


def _sc_aggregate(x_hbm, src_hbm, dst_hbm, acc_out, cnt_out,
                  acc_sh, cnt_sh, rows_v, sidx_v, didx_v, ones_v, cbuf_v,
                  gsem0, gsem1, ssem0, ssem1, csem0, csem1):
    cid = lax.axis_index("c")
    sid = lax.axis_index("s")
    wid = cid * NS + sid
    base_w = pl.multiple_of(wid * EPW, CHUNK)
    bufs = ((0, gsem0, ssem0, csem0), (1, gsem1, ssem1, csem1))

    def zrow(r, _):
        for c in range(D // 16):
            rows_v[0, 0, r, pl.ds(c * 16, 16)] = jnp.zeros((16,), jnp.float32)
        return 0
    lax.fori_loop(0, CHUNK, zrow, 0)
    _zero_f32(cbuf_v, RPT)

    def zones(i, _):
        ones_v[pl.ds(i * 16, 16)] = jnp.ones((16,), jnp.float32)
        return 0
    lax.fori_loop(0, CHUNK // 16, zones, 0)

    for k in range(RPT // CHUNK):
        pltpu.sync_copy(rows_v.at[0, 0],
                        acc_sh.at[pl.ds(sid * RPT + k * CHUNK, CHUNK)])
    pltpu.sync_copy(cbuf_v, cnt_sh.at[pl.ds(sid * RPT, RPT)])
    plsc.subcore_barrier()

    def fire_gathers(g, b, gs):
        for k in range(K):
            idx = sidx_v.at[pl.ds((g * K + k) * CHUNK, CHUNK)]
            pltpu.async_copy(x_hbm.at[idx], rows_v.at[b, k], gs)

    def wait_gathers(b, gs):
        for k in range(K):
            pltpu.make_async_copy(x_hbm.at[pl.ds(0, CHUNK)],
                                  rows_v.at[b, k], gs).wait()

    def do_scatters(g, b, ss, cs):
        scat = []
        for k in range(K):
            didx = didx_v.at[g * K + k]
            scat.append(pltpu.async_copy(rows_v.at[b, k], acc_sh.at[didx], ss,
                                         add=True))
            scat.append(pltpu.async_copy(ones_v, cnt_sh.at[didx], cs,
                                         add=True))
        for cp in scat:
            cp.wait()

    G = NCH // K
    T = G // NB

    def pipe_body(t, _):
        for b, gs, ss, cs in bufs:
            g = NB * t + b
            wait_gathers(b, gs)
            do_scatters(g, b, ss, cs)
            fire_gathers(g + NB, b, gs)
        return 0

    for half in range(HALVES):
        pltpu.sync_copy(
            src_hbm.at[pl.ds(base_w + half * (EPW // HALVES), EPW // HALVES)],
            sidx_v)
        pltpu.sync_copy(
            dst_hbm.at[pl.ds(wid * NCHUNK + half * NCH, NCH)], didx_v)
        for b, gs, _, _ in bufs:
            fire_gathers(b, b, gs)
        lax.fori_loop(0, T - 1, pipe_body, 0)
        for b, gs, ss, cs in bufs:
            wait_gathers(b, gs)
            do_scatters(G - NB + b, b, ss, cs)

    plsc.subcore_barrier()

    pltpu.sync_copy(acc_sh.at[pl.ds(sid * RPT, RPT)],
                    acc_out.at[pl.ds(cid * NPAD + sid * RPT, RPT)])
    pltpu.sync_copy(cnt_sh.at[pl.ds(sid * RPT, RPT)],
                    cnt_out.at[pl.ds(cid * NPAD + sid * RPT, RPT)])


_sc_agg_call = functools.partial(
    pl.kernel,
    out_type=(
        jax.ShapeDtypeStruct((NC * NPAD, D), jnp.float32),
        jax.ShapeDtypeStruct((NC * NPAD,), jnp.float32),
    ),
    mesh=plsc.VectorSubcoreMesh(
        core_axis_name="c", subcore_axis_name="s", num_cores=NC, num_subcores=NS
    ),
    scratch_types=[
        pltpu.VMEM_SHARED((NPAD, D), jnp.float32),
        pltpu.VMEM_SHARED((NPAD,), jnp.float32),
        pltpu.VMEM((NB, K, CHUNK, D), jnp.float32),
        pltpu.VMEM((EPW // HALVES,), jnp.int32),
        pltpu.VMEM((NCH, CHUNK), jnp.int32),
        pltpu.VMEM((CHUNK,), jnp.float32),
        pltpu.VMEM((RPT,), jnp.float32),
        pltpu.SemaphoreType.DMA,
        pltpu.SemaphoreType.DMA,
        pltpu.SemaphoreType.DMA,
        pltpu.SemaphoreType.DMA,
        pltpu.SemaphoreType.DMA,
        pltpu.SemaphoreType.DMA,
    ],
)(_sc_aggregate)


def _tail_body(acc_ref, cnt_ref, x_ref, wi_ref, bi_ref, wl_ref, bl_ref,
               wr_ref, nw_ref, nb_ref, alpha_ref, wo_ref, bo_ref, o_ref):
    p = acc_ref[0:N, :] + acc_ref[NPAD:NPAD + N, :]
    c = cnt_ref[0:N, :] + cnt_ref[NPAD:NPAD + N, :]
    cc = jnp.clip(c, 1.0, None)
    meanx = p / cc
    mask = c / cc
    mm = lambda a, b: lax.dot_general(a, b, (((1,), (1,)), ((), ())),
                                      preferred_element_type=jnp.float32)
    w1 = mm(wl_ref[...], wi_ref[...].T)
    w2 = mm(wr_ref[...], wi_ref[...].T)
    bi = bi_ref[...]
    h2 = (mm(meanx, w1) + mm(x_ref[0:N, :], w2)
          + mask * mm(bi, wl_ref[...]) + mm(bi, wr_ref[...])
          + bl_ref[...])
    mu = jnp.mean(h2, axis=0, keepdims=True)
    centered = h2 - alpha_ref[...] * mu
    var = jnp.mean(centered * centered, axis=0, keepdims=True)
    hn = nw_ref[...] * (centered * lax.rsqrt(var + 1e-5)) + nb_ref[...]
    ha = jnp.where(hn > 0, hn, 0.1 * hn)
    o_ref[...] = mm(ha, wo_ref[...]) + bo_ref[...]


def kernel(x, edge_index, fc_in_w, fc_in_b, lin_l_w, lin_l_b, lin_r_w,
           norm_weight, norm_bias, norm_mean_scale, fc_out_w, fc_out_b):
    f32 = jnp.float32
    xp = jnp.pad(x, ((0, NPAD - N), (0, 0)))
    npad_e = EPAD - E
    sent = (N + (jnp.arange(npad_e, dtype=jnp.int32) % (NPAD - N))).astype(jnp.int32)
    src = jnp.concatenate([edge_index[0], sent])
    dst = jnp.concatenate([edge_index[1], sent]).reshape(EPAD // CHUNK, CHUNK)

    acc, cnt = _sc_agg_call(xp, src, dst)

    out = pl.pallas_call(
        _tail_body,
        out_shape=jax.ShapeDtypeStruct((N, D), f32),
    )(acc, cnt[:, None], xp, fc_in_w, fc_in_b[None, :], lin_l_w,
      lin_l_b[None, :], lin_r_w, norm_weight[None, :], norm_bias[None, :],
      norm_mean_scale[None, :], fc_out_w, fc_out_b[None, :])
    return out


# speedup vs baseline: 12.7082x; 1.0607x over previous
import functools

import jax
import jax.numpy as jnp
from jax import lax
from jax.experimental import pallas as pl
from jax.experimental.pallas import tpu as pltpu
from jax.experimental.pallas import tpu_sc as plsc

N = 10000
D = 128
NPAD = 10240
E = 320000
NC, NS = 2, 16
NW = NC * NS
CHUNK = 128
K = 1
NB = 2
HALVES = 2
GRAN = CHUNK * K * HALVES
EPW = ((E // NW + GRAN - 1) // GRAN) * GRAN
EPAD = EPW * NW
NCHUNK = EPW // CHUNK
NCH = NCHUNK // HALVES
RPT = NPAD // NS


def _zero_f32(ref, n):
    def body(i, _):
        ref[pl.ds(i * 16, 16)] = jnp.zeros((16,), jnp.float32)
        return 0
    lax.fori_loop(0, n // 16, body, 0)


def _sc_aggregate(x_hbm, src_hbm, dst_hbm, acc_out, cnt_out,
                  acc_sh, cnt_sh, rows_v, sidx_v, didx_v, ones_v, cbuf_v,
                  gsem0, gsem1, ssem0, ssem1, csem0, csem1):
    cid = lax.axis_index("c")
    sid = lax.axis_index("s")
    wid = cid * NS + sid
    base_w = pl.multiple_of(wid * EPW, CHUNK)
    bufs = ((0, gsem0, ssem0, csem0), (1, gsem1, ssem1, csem1))

    def zrow(r, _):
        for c in range(D // 16):
            rows_v[0, 0, r, pl.ds(c * 16, 16)] = jnp.zeros((16,), jnp.float32)
        return 0
    lax.fori_loop(0, CHUNK, zrow, 0)
    _zero_f32(cbuf_v, RPT)

    def zones(i, _):
        ones_v[pl.ds(i * 16, 16)] = jnp.ones((16,), jnp.float32)
        return 0
    lax.fori_loop(0, CHUNK // 16, zones, 0)

    for k in range(RPT // CHUNK):
        pltpu.sync_copy(rows_v.at[0, 0],
                        acc_sh.at[pl.ds(sid * RPT + k * CHUNK, CHUNK)])
    pltpu.sync_copy(cbuf_v, cnt_sh.at[pl.ds(sid * RPT, RPT)])
    plsc.subcore_barrier()

    def fire_gathers(g, b, gs):
        for k in range(K):
            idx = sidx_v.at[pl.ds((g * K + k) * CHUNK, CHUNK)]
            pltpu.async_copy(x_hbm.at[idx], rows_v.at[b, k], gs)

    def wait_gathers(b, gs):
        for k in range(K):
            pltpu.make_async_copy(x_hbm.at[pl.ds(0, CHUNK)],
                                  rows_v.at[b, k], gs).wait()

    def do_scatters(g, b, ss, cs):
        scat = []
        for k in range(K):
            didx = didx_v.at[g * K + k]
            scat.append(pltpu.async_copy(rows_v.at[b, k], acc_sh.at[didx], ss,
                                         add=True))
            scat.append(pltpu.async_copy(ones_v, cnt_sh.at[didx], cs,
                                         add=True))
        for cp in scat:
            cp.wait()

    G = NCH // K
    T = G // NB

    def pipe_body(t, _):
        for b, gs, ss, cs in bufs:
            g = NB * t + b
            wait_gathers(b, gs)
            do_scatters(g, b, ss, cs)
            fire_gathers(g + NB, b, gs)
        return 0

    for half in range(HALVES):
        pltpu.sync_copy(
            src_hbm.at[pl.ds(base_w + half * (EPW // HALVES), EPW // HALVES)],
            sidx_v)
        pltpu.sync_copy(
            dst_hbm.at[pl.ds(wid * NCHUNK + half * NCH, NCH)], didx_v)
        for b, gs, _, _ in bufs:
            fire_gathers(b, b, gs)
        lax.fori_loop(0, T - 1, pipe_body, 0)
        for b, gs, ss, cs in bufs:
            wait_gathers(b, gs)
            do_scatters(G - NB + b, b, ss, cs)

    plsc.subcore_barrier()

    pltpu.sync_copy(acc_sh.at[pl.ds(sid * RPT, RPT)],
                    acc_out.at[pl.ds(cid * NPAD + sid * RPT, RPT)])
    pltpu.sync_copy(cnt_sh.at[pl.ds(sid * RPT, RPT)],
                    cnt_out.at[pl.ds(cid * NPAD + sid * RPT, RPT)])


_sc_agg_call = functools.partial(
    pl.kernel,
    out_type=(
        jax.ShapeDtypeStruct((NC * NPAD, D), jnp.float32),
        jax.ShapeDtypeStruct((NC * NPAD,), jnp.float32),
    ),
    mesh=plsc.VectorSubcoreMesh(
        core_axis_name="c", subcore_axis_name="s", num_cores=NC, num_subcores=NS
    ),
    scratch_types=[
        pltpu.VMEM_SHARED((NPAD, D), jnp.float32),
        pltpu.VMEM_SHARED((NPAD,), jnp.float32),
        pltpu.VMEM((NB, K, CHUNK, D), jnp.float32),
        pltpu.VMEM((EPW // HALVES,), jnp.int32),
        pltpu.VMEM((NCH, CHUNK), jnp.int32),
        pltpu.VMEM((CHUNK,), jnp.float32),
        pltpu.VMEM((RPT,), jnp.float32),
        pltpu.SemaphoreType.DMA,
        pltpu.SemaphoreType.DMA,
        pltpu.SemaphoreType.DMA,
        pltpu.SemaphoreType.DMA,
        pltpu.SemaphoreType.DMA,
        pltpu.SemaphoreType.DMA,
    ],
)(_sc_aggregate)


def _tail_body(acc_ref, cnt_ref, x_ref, wi_ref, bi_ref, wl_ref, bl_ref,
               wr_ref, nw_ref, nb_ref, alpha_ref, wo_ref, bo_ref, o_ref):
    p = acc_ref[0:N, :] + acc_ref[NPAD:NPAD + N, :]
    c = cnt_ref[0:N, :] + cnt_ref[NPAD:NPAD + N, :]
    cc = jnp.clip(c, 1.0, None)
    meanx = p / cc
    mask = c / cc
    mm = lambda a, b: lax.dot_general(a, b, (((1,), (1,)), ((), ())),
                                      preferred_element_type=jnp.float32)
    w1 = mm(wl_ref[...], wi_ref[...].T)
    w2 = mm(wr_ref[...], wi_ref[...].T)
    bi = bi_ref[...]
    h2 = (mm(meanx, w1) + mm(x_ref[0:N, :], w2)
          + mask * mm(bi, wl_ref[...]) + mm(bi, wr_ref[...])
          + bl_ref[...])
    mu = jnp.mean(h2, axis=0, keepdims=True)
    centered = h2 - alpha_ref[...] * mu
    var = jnp.mean(centered * centered, axis=0, keepdims=True)
    hn = nw_ref[...] * (centered * lax.rsqrt(var + 1e-5)) + nb_ref[...]
    ha = jnp.where(hn > 0, hn, 0.1 * hn)
    o_ref[...] = mm(ha, wo_ref[...]) + bo_ref[...]


def kernel(x, edge_index, fc_in_w, fc_in_b, lin_l_w, lin_l_b, lin_r_w,
           norm_weight, norm_bias, norm_mean_scale, fc_out_w, fc_out_b):
    f32 = jnp.float32
    xp = jnp.pad(x, ((0, NPAD - N), (0, 0)))
    npad_e = EPAD - E
    sent = (N + (jnp.arange(npad_e, dtype=jnp.int32) % (NPAD - N))).astype(jnp.int32)
    src = jnp.concatenate([edge_index[0], sent])
    dst = jnp.concatenate([edge_index[1], sent]).reshape(EPAD // CHUNK, CHUNK)

    acc, cnt = _sc_agg_call(xp, src, dst)

    out = pl.pallas_call(
        _tail_body,
        out_shape=jax.ShapeDtypeStruct((N, D), f32),
    )(acc, cnt[:, None], xp, fc_in_w, fc_in_b[None, :], lin_l_w,
      lin_l_b[None, :], lin_r_w, norm_weight[None, :], norm_bias[None, :],
      norm_mean_scale[None, :], fc_out_w, fc_out_b[None, :])
    return out


# speedup vs baseline: 12.8205x; 1.0088x over previous
import functools

import jax
import jax.numpy as jnp
from jax import lax
from jax.experimental import pallas as pl
from jax.experimental.pallas import tpu as pltpu
from jax.experimental.pallas import tpu_sc as plsc

N = 10000
D = 128
NPAD = 10240
E = 320000
NC, NS = 2, 16
NW = NC * NS
CHUNK = 128
K = 1
NB = 2
HALVES = 2
GRAN = CHUNK * K * HALVES
EPW = ((E // NW + GRAN - 1) // GRAN) * GRAN
EPAD = EPW * NW
NCHUNK = EPW // CHUNK
NCH = NCHUNK // HALVES
RPT = NPAD // NS


def _zero_f32(ref, n):
    def body(i, _):
        ref[pl.ds(i * 16, 16)] = jnp.zeros((16,), jnp.float32)
        return 0
    lax.fori_loop(0, n // 16, body, 0)


def _sc_aggregate(x_hbm, src_hbm, dst_hbm, acc_out, cnt_out,
                  acc_sh, cnt_sh, rows_v, sidx_v, didx_v, ones_v, cbuf_v,
                  gsem0, gsem1, ssem0, ssem1, csem0, csem1):
    cid = lax.axis_index("c")
    sid = lax.axis_index("s")
    wid = cid * NS + sid
    base_w = pl.multiple_of(wid * EPW, CHUNK)
    bufs = ((0, gsem0, ssem0, csem0), (1, gsem1, ssem1, csem1))

    def zrow(r, _):
        for c in range(D // 16):
            rows_v[0, 0, r, pl.ds(c * 16, 16)] = jnp.zeros((16,), jnp.float32)
        return 0
    lax.fori_loop(0, CHUNK, zrow, 0)
    _zero_f32(cbuf_v, RPT)

    def zones(i, _):
        ones_v[pl.ds(i * 16, 16)] = jnp.ones((16,), jnp.float32)
        return 0
    lax.fori_loop(0, CHUNK // 16, zones, 0)

    for k in range(RPT // CHUNK):
        pltpu.sync_copy(rows_v.at[0, 0],
                        acc_sh.at[pl.ds(sid * RPT + k * CHUNK, CHUNK)])
    pltpu.sync_copy(cbuf_v, cnt_sh.at[pl.ds(sid * RPT, RPT)])
    plsc.subcore_barrier()

    def fire_gathers(g, b, gs):
        for k in range(K):
            idx = sidx_v.at[pl.ds((g * K + k) * CHUNK, CHUNK)]
            pltpu.async_copy(x_hbm.at[idx], rows_v.at[b, k], gs)

    def wait_gathers(b, gs):
        for k in range(K):
            pltpu.make_async_copy(x_hbm.at[pl.ds(0, CHUNK)],
                                  rows_v.at[b, k], gs).wait()

    def do_scatters(g, b, ss, cs):
        scat = []
        for k in range(K):
            didx = didx_v.at[g * K + k]
            scat.append(pltpu.async_copy(rows_v.at[b, k], acc_sh.at[didx], ss,
                                         add=True))
            scat.append(pltpu.async_copy(ones_v, cnt_sh.at[didx], cs,
                                         add=True))
        for cp in scat:
            cp.wait()

    G = NCH // K
    T = G // NB

    def pipe_body(t, _):
        for b, gs, ss, cs in bufs:
            g = NB * t + b
            wait_gathers(b, gs)
            do_scatters(g, b, ss, cs)
            fire_gathers(g + NB, b, gs)
        return 0

    for half in range(HALVES):
        pltpu.sync_copy(
            src_hbm.at[pl.ds(base_w + half * (EPW // HALVES), EPW // HALVES)],
            sidx_v)
        pltpu.sync_copy(
            dst_hbm.at[pl.ds(wid * NCHUNK + half * NCH, NCH)], didx_v)
        for b, gs, _, _ in bufs:
            fire_gathers(b, b, gs)
        lax.fori_loop(0, T - 1, pipe_body, 0)
        for b, gs, ss, cs in bufs:
            wait_gathers(b, gs)
            do_scatters(G - NB + b, b, ss, cs)

    plsc.subcore_barrier()

    pltpu.sync_copy(acc_sh.at[pl.ds(sid * RPT, RPT)],
                    acc_out.at[pl.ds(cid * NPAD + sid * RPT, RPT)])
    pltpu.sync_copy(cnt_sh.at[pl.ds(sid * RPT, RPT)],
                    cnt_out.at[pl.ds(cid * NPAD + sid * RPT, RPT)])


_sc_agg_call = functools.partial(
    pl.kernel,
    out_type=(
        jax.ShapeDtypeStruct((NC * NPAD, D), jnp.float32),
        jax.ShapeDtypeStruct((NC * NPAD,), jnp.float32),
    ),
    mesh=plsc.VectorSubcoreMesh(
        core_axis_name="c", subcore_axis_name="s", num_cores=NC, num_subcores=NS
    ),
    scratch_types=[
        pltpu.VMEM_SHARED((NPAD, D), jnp.float32),
        pltpu.VMEM_SHARED((NPAD,), jnp.float32),
        pltpu.VMEM((NB, K, CHUNK, D), jnp.float32),
        pltpu.VMEM((EPW // HALVES,), jnp.int32),
        pltpu.VMEM((NCH, CHUNK), jnp.int32),
        pltpu.VMEM((CHUNK,), jnp.float32),
        pltpu.VMEM((RPT,), jnp.float32),
        pltpu.SemaphoreType.DMA,
        pltpu.SemaphoreType.DMA,
        pltpu.SemaphoreType.DMA,
        pltpu.SemaphoreType.DMA,
        pltpu.SemaphoreType.DMA,
        pltpu.SemaphoreType.DMA,
    ],
)(_sc_aggregate)


def _tail_body(acc_ref, cnt_ref, x_ref, wi_ref, bi_ref, wl_ref, bl_ref,
               wr_ref, nw_ref, nb_ref, alpha_ref, wo_ref, bo_ref, o_ref):
    p = acc_ref[0:N, :] + acc_ref[NPAD:NPAD + N, :]
    c = cnt_ref[0:N, :] + cnt_ref[NPAD:NPAD + N, :]
    cc = jnp.clip(c, 1.0, None)
    meanx = p / cc
    mask = c / cc
    mm = lambda a, b: lax.dot_general(a, b, (((1,), (1,)), ((), ())),
                                      preferred_element_type=jnp.float32)
    w1 = mm(wl_ref[...], wi_ref[...].T)
    w2 = mm(wr_ref[...], wi_ref[...].T)
    bi = bi_ref[...]
    h2 = (mm(meanx, w1) + mm(x_ref[...], w2)
          + mask * mm(bi, wl_ref[...]) + mm(bi, wr_ref[...])
          + bl_ref[...])
    mu = jnp.mean(h2, axis=0, keepdims=True)
    centered = h2 - alpha_ref[...] * mu
    var = jnp.mean(centered * centered, axis=0, keepdims=True)
    hn = nw_ref[...] * (centered * lax.rsqrt(var + 1e-5)) + nb_ref[...]
    ha = jnp.where(hn > 0, hn, 0.1 * hn)
    o_ref[...] = mm(ha, wo_ref[...]) + bo_ref[...]


def kernel(x, edge_index, fc_in_w, fc_in_b, lin_l_w, lin_l_b, lin_r_w,
           norm_weight, norm_bias, norm_mean_scale, fc_out_w, fc_out_b):
    f32 = jnp.float32
    npad_e = EPAD - E
    ar = jnp.arange(npad_e, dtype=jnp.int32)
    src = jnp.concatenate([edge_index[0], ar % N])
    dst = jnp.concatenate([edge_index[1],
                           N + (ar % (NPAD - N))]).reshape(EPAD // CHUNK, CHUNK)

    acc, cnt = _sc_agg_call(x, src, dst)

    out = pl.pallas_call(
        _tail_body,
        out_shape=jax.ShapeDtypeStruct((N, D), f32),
    )(acc, cnt[:, None], x, fc_in_w, fc_in_b[None, :], lin_l_w,
      lin_l_b[None, :], lin_r_w, norm_weight[None, :], norm_bias[None, :],
      norm_mean_scale[None, :], fc_out_w, fc_out_b[None, :])
    return out
